# baseline (speedup 1.0000x reference)
import jax
import jax.numpy as jnp
from jax.experimental import pallas as pl
from jax.experimental.pallas import tpu as pltpu

_CR = 2000
_NBUF = 4
_LANES = 128


def _copy_body(x_hbm, ea_hbm, xo_hbm, eao_hbm, buf, sin, sout):
    segs = []
    for src, dst in ((ea_hbm, eao_hbm), (x_hbm, xo_hbm)):
        for c in range(src.shape[0] // _CR):
            segs.append((src, dst, c * _CR))
    n = len(segs)

    def in_copy(i, slot):
        src, _, r0 = segs[i]
        return pltpu.make_async_copy(
            src.at[pl.ds(r0, _CR), :], buf.at[slot], sin.at[slot])

    def out_copy(i, slot):
        _, dst, r0 = segs[i]
        return pltpu.make_async_copy(
            buf.at[slot], dst.at[pl.ds(r0, _CR), :], sout.at[slot])

    for i in range(min(_NBUF, n)):
        in_copy(i, i).start()
    for i in range(n):
        slot = i % _NBUF
        in_copy(i, slot).wait()
        out_copy(i, slot).start()
        j = i + _NBUF
        if j < n:
            out_copy(i, slot).wait()
            in_copy(j, slot).start()
    for i in range(max(0, n - _NBUF), n):
        out_copy(i, i % _NBUF).wait()


def kernel(x, x_lstm, encoded_z_gnss, edge_index, edge_attr,
           node_indexes_related_to_agent, edge_indexes_related_to_agent):
    N, DF = x.shape
    E, DE = edge_attr.shape
    ER = (E * DE) // _LANES
    ea = edge_attr.reshape(ER, _LANES)
    xn, ean = pl.pallas_call(
        _copy_body,
        in_specs=[
            pl.BlockSpec(memory_space=pl.ANY),
            pl.BlockSpec(memory_space=pl.ANY),
        ],
        out_specs=[
            pl.BlockSpec(memory_space=pl.ANY),
            pl.BlockSpec(memory_space=pl.ANY),
        ],
        out_shape=[
            jax.ShapeDtypeStruct((N, DF), x.dtype),
            jax.ShapeDtypeStruct((ER, _LANES), edge_attr.dtype),
        ],
        scratch_shapes=[
            pltpu.VMEM((_NBUF, _CR, _LANES), jnp.float32),
            pltpu.SemaphoreType.DMA((_NBUF,)),
            pltpu.SemaphoreType.DMA((_NBUF,)),
        ],
    )(x, ea)
    return (xn, ean.reshape(E, DE))


# speedup vs baseline: 1.0317x; 1.0317x over previous
import jax
import jax.numpy as jnp
from jax.experimental import pallas as pl
from jax.experimental.pallas import tpu as pltpu

_CR = 2000
_NBUF = 25
_LANES = 128


def _copy_body(x_hbm, ea_hbm, xo_hbm, eao_hbm, buf, sin, sout):
    segs = []
    for src, dst in ((ea_hbm, eao_hbm), (x_hbm, xo_hbm)):
        for c in range(src.shape[0] // _CR):
            segs.append((src, dst, c * _CR))
    n = len(segs)

    def in_copy(i, slot):
        src, _, r0 = segs[i]
        return pltpu.make_async_copy(
            src.at[pl.ds(r0, _CR), :], buf.at[slot], sin.at[slot])

    def out_copy(i, slot):
        _, dst, r0 = segs[i]
        return pltpu.make_async_copy(
            buf.at[slot], dst.at[pl.ds(r0, _CR), :], sout.at[slot])

    for i in range(n):
        in_copy(i, i).start()
    for i in range(n):
        in_copy(i, i).wait()
        out_copy(i, i).start()
    for i in range(n):
        out_copy(i, i).wait()


def kernel(x, x_lstm, encoded_z_gnss, edge_index, edge_attr,
           node_indexes_related_to_agent, edge_indexes_related_to_agent):
    N, DF = x.shape
    E, DE = edge_attr.shape
    ER = (E * DE) // _LANES
    ea = edge_attr.reshape(ER, _LANES)
    xn, ean = pl.pallas_call(
        _copy_body,
        in_specs=[
            pl.BlockSpec(memory_space=pl.ANY),
            pl.BlockSpec(memory_space=pl.ANY),
        ],
        out_specs=[
            pl.BlockSpec(memory_space=pl.ANY),
            pl.BlockSpec(memory_space=pl.ANY),
        ],
        out_shape=[
            jax.ShapeDtypeStruct((N, DF), x.dtype),
            jax.ShapeDtypeStruct((ER, _LANES), edge_attr.dtype),
        ],
        scratch_shapes=[
            pltpu.VMEM((_NBUF, _CR, _LANES), jnp.float32),
            pltpu.SemaphoreType.DMA((_NBUF,)),
            pltpu.SemaphoreType.DMA((_NBUF,)),
        ],
    )(x, ea)
    return (xn, ean.reshape(E, DE))
